# baseline (speedup 1.0000x reference)
import functools
import jax
import jax.numpy as jnp
from jax import lax
from jax.experimental import pallas as pl
from jax.experimental.pallas import tpu as pltpu
from jax.experimental.pallas import tpu_sc as plsc

_N = 10000
_B = 8
_F = 128
_H = 32
_P = 4
_E = 320000
_NPAD = 10240
_DUMMY = 10000
_NC = 2
_NS = 16
_K = 80
_ASTEPS = _E // (_NC * _NS) // _K
_CSTEPS = _E // _NS // _K
_WSTEPS = 25
_NWIN = _CSTEPS // _WSTEPS
_AWIN = _ASTEPS // _WSTEPS
_TILEN = 2000
_NBLK = _N // _TILEN


def _for(n, f):
    def body(i, carry):
        f(i)
        return carry
    lax.fori_loop(0, n, body, 0)


def _deg_sc_body(rows_hbm, cols_hbm, deg_out, r2, c2, ones_v, stage, acc):
    core = lax.axis_index("c")
    tile = lax.axis_index("s")
    wid = core * _NS + tile
    row0 = tile * 640

    def fill(i):
        s = i // 8
        j = i % 8
        ones_v[s, pl.ds(j * 16, 16)] = jnp.full((16,), 1.0, jnp.float32)
        stage[s, pl.ds(j * 16, 16)] = jnp.zeros((16,), jnp.float32)
    _for(_K * 8, fill)

    def zero(m):
        pltpu.sync_copy(stage, acc.at[pl.ds(row0 + m * _K, _K)])
    _for(8, zero)
    plsc.subcore_barrier()

    def win(w):
        pltpu.sync_copy(rows_hbm.at[wid, w], r2)
        pltpu.sync_copy(cols_hbm.at[wid, w], c2)

        def mask(i):
            s = i // 5
            j = i % 5
            r = r2[s, pl.ds(j * 16, 16)]
            c = c2[s, pl.ds(j * 16, 16)]
            c2[s, pl.ds(j * 16, 16)] = jnp.where(r == c, _DUMMY, c)
        _for(_WSTEPS * 5, mask)

        def step(s):
            pltpu.sync_copy(ones_v, acc.at[c2.at[s]], add=True)
        _for(_WSTEPS, step)
    _for(_AWIN, win)

    plsc.subcore_barrier()

    def dump(m):
        pltpu.sync_copy(acc.at[pl.ds(row0 + m * _K, _K)], stage)
        pltpu.sync_copy(stage, deg_out.at[core, pl.ds(row0 + m * _K, _K)])
    _for(8, dump)


def _prep_body(xt_r, dp_r, w_r, up_r, dv_r):
    deg = dp_r[0, :, 0] + dp_r[1, :, 0] + 1.0
    dv = lax.rsqrt(deg)
    x = xt_r[0]
    us = []
    for p in range(_P):
        u = jnp.dot(x[:, p, :], w_r[...], preferred_element_type=jnp.float32)
        us.append(u * dv[:, None])
    up_r[0, 0] = jnp.concatenate([us[0], us[1]], axis=1)
    up_r[0, 1] = jnp.concatenate([us[2], us[3]], axis=1)
    dv_r[0, 0] = dv


_prep_tc = pl.pallas_call(
    _prep_body,
    grid=(_B, _NBLK),
    in_specs=[
        pl.BlockSpec((1, _TILEN, _P, _F), lambda b, j: (b, j, 0, 0)),
        pl.BlockSpec((_NC, _TILEN, _F), lambda b, j: (0, j, 0)),
        pl.BlockSpec((_F, 2 * _H), lambda b, j: (0, 0)),
    ],
    out_specs=[
        pl.BlockSpec((1, 2, _TILEN, _F), lambda b, j: (b, 0, j, 0)),
        pl.BlockSpec((1, 1, _TILEN), lambda b, j: (j, 0, 0)),
    ],
    out_shape=[
        jax.ShapeDtypeStruct((_B, 2, _N, _F), jnp.float32),
        jax.ShapeDtypeStruct((_NBLK, 1, _TILEN), jnp.float32),
    ],
)


def _scatter_sc_body(rows_hbm, cols_hbm, table_hbm, s_out,
                     r2, c2, idx0, idx1, msg0, msg1, acc,
                     sem0, sem1, ssem0, ssem1):
    core = lax.axis_index("c")
    tile = lax.axis_index("s")
    row0 = tile * 640
    msg = msg0

    def one_pass(pi):
        chunk = core * 8 + pi
        base = chunk * _N

        def zfill(i):
            s = i // 8
            j = i % 8
            msg[s, pl.ds(j * 16, 16)] = jnp.zeros((16,), jnp.float32)
        _for(_K * 8, zfill)

        def zero(m):
            pltpu.sync_copy(msg, acc.at[pl.ds(row0 + m * _K, _K)])
        _for(8, zero)
        plsc.subcore_barrier()

        def win(w):
            pltpu.sync_copy(rows_hbm.at[tile, w], r2)
            pltpu.sync_copy(cols_hbm.at[tile, w], c2)

            def mask(i):
                s = i // 5
                j = i % 5
                r = r2[s, pl.ds(j * 16, 16)]
                c = c2[s, pl.ds(j * 16, 16)]
                c2[s, pl.ds(j * 16, 16)] = jnp.where(r == c, _DUMMY, c)
            _for(_WSTEPS * 5, mask)

            def start_g(s, ib, mb, gsem):
                def fill(j):
                    ib[pl.ds(j * 16, 16)] = r2[s, pl.ds(j * 16, 16)] + base
                _for(5, fill)
                pltpu.async_copy(table_hbm.at[ib], mb, gsem)

            def wait_g(ib, mb, gsem):
                pltpu.make_async_copy(table_hbm.at[ib], mb, gsem).wait()

            def issue_s(s, mb, ssem):
                pltpu.async_copy(mb, acc.at[c2.at[s]], ssem, add=True)

            def wait_s(s, mb, ssem):
                pltpu.make_async_copy(mb, acc.at[c2.at[s]], ssem).wait()

            start_g(0, idx0, msg0, sem0)
            start_g(1, idx1, msg1, sem1)

            def pair(t):
                s0 = 2 * t
                wait_g(idx0, msg0, sem0)
                issue_s(s0, msg0, ssem0)
                wait_g(idx1, msg1, sem1)
                issue_s(s0 + 1, msg1, ssem1)
                wait_s(s0, msg0, ssem0)
                start_g(s0 + 2, idx0, msg0, sem0)
                wait_s(s0 + 1, msg1, ssem1)
                start_g(s0 + 3, idx1, msg1, sem1)
            _for((_WSTEPS - 3) // 2, pair)
            sl = _WSTEPS - 3
            wait_g(idx0, msg0, sem0)
            issue_s(sl, msg0, ssem0)
            wait_g(idx1, msg1, sem1)
            issue_s(sl + 1, msg1, ssem1)
            wait_s(sl, msg0, ssem0)
            start_g(sl + 2, idx0, msg0, sem0)
            wait_s(sl + 1, msg1, ssem1)
            wait_g(idx0, msg0, sem0)
            issue_s(sl + 2, msg0, ssem0)
            wait_s(sl + 2, msg0, ssem0)
        _for(_NWIN, win)
        plsc.subcore_barrier()

        def dump(m):
            pltpu.sync_copy(acc.at[pl.ds(row0 + m * _K, _K)], msg)
            pltpu.sync_copy(msg, s_out.at[chunk, pl.ds(row0 + m * _K, _K)])
        _for(8, dump)
    _for(8, one_pass)


def _gates_body(sp_r, up_r, dv_r, lzw_r, lzb_r, lhw_r, lhb_r,
                bz_r, bh_r, att_r, wl_r, bl_r, out_r):
    dv = dv_r[0, 0]
    S = dv[None, :, None] * (sp_r[0] + up_r[0])
    am = att_r[...]
    e = jnp.exp(am - jnp.max(am))
    probs = e / jnp.sum(e)
    acc = jnp.zeros((_TILEN, _H), jnp.float32)
    for p in range(_P):
        half, pi = p // 2, p % 2
        fs = S[half][:, pi * 64:(pi + 1) * 64]
        cz = fs[:, :_H] + bz_r[...]
        ch = fs[:, _H:] + bh_r[...]
        Z = jax.nn.sigmoid(
            jnp.dot(cz, lzw_r[: _H, :], preferred_element_type=jnp.float32)
            + lzb_r[...])
        Ht = jnp.tanh(
            jnp.dot(ch, lhw_r[: _H, :], preferred_element_type=jnp.float32)
            + lhb_r[...])
        acc = acc + probs[p:p + 1] * ((1.0 - Z) * Ht)
    res = jnp.dot(jnp.maximum(acc, 0.0), wl_r[...],
                  preferred_element_type=jnp.float32) + bl_r[...]
    out_r[0] = res


_gates_tc = pl.pallas_call(
    _gates_body,
    grid=(_B, _NBLK),
    in_specs=[
        pl.BlockSpec((1, 2, _TILEN, _F), lambda b, j: (b, 0, j, 0)),
        pl.BlockSpec((1, 2, _TILEN, _F), lambda b, j: (b, 0, j, 0)),
        pl.BlockSpec((1, 1, _TILEN), lambda b, j: (j, 0, 0)),
        pl.BlockSpec((2 * _H, _H), lambda b, j: (0, 0)),
        pl.BlockSpec((_H,), lambda b, j: (0,)),
        pl.BlockSpec((2 * _H, _H), lambda b, j: (0, 0)),
        pl.BlockSpec((_H,), lambda b, j: (0,)),
        pl.BlockSpec((_H,), lambda b, j: (0,)),
        pl.BlockSpec((_H,), lambda b, j: (0,)),
        pl.BlockSpec((_P,), lambda b, j: (0,)),
        pl.BlockSpec((_H, _P), lambda b, j: (0, 0)),
        pl.BlockSpec((_P,), lambda b, j: (0,)),
    ],
    out_specs=pl.BlockSpec((1, _TILEN, _P), lambda b, j: (b, j, 0)),
    out_shape=jax.ShapeDtypeStruct((_B, _N, _P), jnp.float32),
)


@functools.cache
def _build_sc_kernels():
    mesh = plsc.VectorSubcoreMesh(
        core_axis_name="c", subcore_axis_name="s",
        num_cores=_NC, num_subcores=_NS)
    deg = pl.kernel(
        _deg_sc_body,
        out_type=jax.ShapeDtypeStruct((_NC, _NPAD, _F), jnp.float32),
        mesh=mesh,
        scratch_types=[
            pltpu.VMEM((_WSTEPS, _K), jnp.int32),
            pltpu.VMEM((_WSTEPS, _K), jnp.int32),
            pltpu.VMEM((_K, _F), jnp.float32),
            pltpu.VMEM((_K, _F), jnp.float32),
            pltpu.VMEM_SHARED((_NPAD, _F), jnp.float32),
        ],
    )
    scatter = pl.kernel(
        _scatter_sc_body,
        out_type=jax.ShapeDtypeStruct((2 * _B, _NPAD, _F), jnp.float32),
        mesh=mesh,
        scratch_types=[
            pltpu.VMEM((_WSTEPS, _K), jnp.int32),
            pltpu.VMEM((_WSTEPS, _K), jnp.int32),
            pltpu.VMEM((_K,), jnp.int32),
            pltpu.VMEM((_K,), jnp.int32),
            pltpu.VMEM((_K, _F), jnp.float32),
            pltpu.VMEM((_K, _F), jnp.float32),
            pltpu.VMEM_SHARED((_NPAD, _F), jnp.float32),
            pltpu.SemaphoreType.DMA,
            pltpu.SemaphoreType.DMA,
            pltpu.SemaphoreType.DMA,
            pltpu.SemaphoreType.DMA,
        ],
    )
    return deg, scatter


def kernel(x, edge_index, Wz, bz, Wr, br, Wh, bh,
           LzW, Lzb, LrW, Lrb, LhW, Lhb, att, Wl, bl):
    _deg_sc, _scatter_sc = _build_sc_kernels()
    row = edge_index[0]
    col = edge_index[1]
    rows_a = row.reshape(_NC * _NS, _AWIN, _WSTEPS, _K)
    cols_a = col.reshape(_NC * _NS, _AWIN, _WSTEPS, _K)
    rows_c = row.reshape(_NS, _NWIN, _WSTEPS, _K)
    cols_c = col.reshape(_NS, _NWIN, _WSTEPS, _K)
    xt = jnp.transpose(x, (0, 1, 3, 2))
    Wcat = jnp.concatenate([Wz, Wh], axis=1)

    dp = _deg_sc(rows_a, cols_a)
    Up, dinv = _prep_tc(xt, dp, Wcat)
    table = Up.reshape(2 * _B * _N, _F)
    Sp = _scatter_sc(rows_c, cols_c, table)
    Sp4 = Sp.reshape(_B, 2, _NPAD, _F)
    out = _gates_tc(Sp4, Up, dinv, LzW, Lzb, LhW, Lhb,
                    bz, bh, att, Wl, bl)
    return out


# speedup vs baseline: 1.2903x; 1.2903x over previous
import functools
import jax
import jax.numpy as jnp
from jax import lax
from jax.experimental import pallas as pl
from jax.experimental.pallas import tpu as pltpu
from jax.experimental.pallas import tpu_sc as plsc

_N = 10000
_B = 8
_F = 128
_H = 32
_P = 4
_E = 320000
_NPAD = 10240
_DUMMY = 10000
_NC = 2
_NS = 16
_K = 80
_ASTEPS = _E // (_NC * _NS) // _K
_CSTEPS = _E // _NS // _K
_WSTEPS = 50
_NWIN = _CSTEPS // _WSTEPS
_AWSTEPS = 25
_AWIN = _ASTEPS // _AWSTEPS
_TILEN = 2000
_NBLK = _N // _TILEN


def _for(n, f):
    def body(i, carry):
        f(i)
        return carry
    lax.fori_loop(0, n, body, 0)


def _deg_sc_body(rows_hbm, cols_hbm, deg_out, r2, c2, ones_v, stage, acc):
    core = lax.axis_index("c")
    tile = lax.axis_index("s")
    wid = core * _NS + tile
    row0 = tile * 640

    def fill(i):
        s = i // 2
        j = i % 2
        ones_v[s, pl.ds(j * 16, 16)] = jnp.full((16,), 1.0, jnp.float32)
        stage[s, pl.ds(j * 16, 16)] = jnp.zeros((16,), jnp.float32)
    _for(_K * 2, fill)

    def zero(m):
        pltpu.sync_copy(stage, acc.at[pl.ds(row0 + m * _K, _K)])
    _for(8, zero)
    plsc.subcore_barrier()

    def win(w):
        pltpu.sync_copy(rows_hbm.at[wid, w], r2)
        pltpu.sync_copy(cols_hbm.at[wid, w], c2)

        def mask(i):
            s = i // 5
            j = i % 5
            r = r2[s, pl.ds(j * 16, 16)]
            c = c2[s, pl.ds(j * 16, 16)]
            c2[s, pl.ds(j * 16, 16)] = jnp.where(r == c, _DUMMY, c)
        _for(_AWSTEPS * 5, mask)

        def step(s):
            pltpu.sync_copy(ones_v, acc.at[c2.at[s]], add=True)
        _for(_AWSTEPS, step)
    _for(_AWIN, win)

    plsc.subcore_barrier()

    def dump(m):
        pltpu.sync_copy(acc.at[pl.ds(row0 + m * _K, _K)], stage)
        pltpu.sync_copy(stage, deg_out.at[core, pl.ds(row0 + m * _K, _K)])
    _for(8, dump)


def _prep_body(xt_r, dp_r, w_r, up_r, dv_r):
    deg = dp_r[0, :, 0] + dp_r[1, :, 0] + 1.0
    dv = lax.rsqrt(deg)
    x = xt_r[0]
    us = []
    for p in range(_P):
        u = jnp.dot(x[:, p, :], w_r[...], preferred_element_type=jnp.float32)
        us.append(u * dv[:, None])
    up_r[0, 0] = jnp.concatenate([us[0], us[1]], axis=1)
    up_r[0, 1] = jnp.concatenate([us[2], us[3]], axis=1)
    dv_r[0, 0] = dv


_prep_tc = pl.pallas_call(
    _prep_body,
    grid=(_B, _NBLK),
    in_specs=[
        pl.BlockSpec((1, _TILEN, _P, _F), lambda b, j: (b, j, 0, 0)),
        pl.BlockSpec((_NC, _TILEN, 32), lambda b, j: (0, j, 0)),
        pl.BlockSpec((_F, 2 * _H), lambda b, j: (0, 0)),
    ],
    out_specs=[
        pl.BlockSpec((1, 2, _TILEN, _F), lambda b, j: (b, 0, j, 0)),
        pl.BlockSpec((1, 1, _TILEN), lambda b, j: (j, 0, 0)),
    ],
    out_shape=[
        jax.ShapeDtypeStruct((_B, 2, _N, _F), jnp.float32),
        jax.ShapeDtypeStruct((_NBLK, 1, _TILEN), jnp.float32),
    ],
)


def _scatter_sc_body(rows_hbm, cols_hbm, table_hbm, s_out,
                     r2, c2, idx0, idx1, msg0, msg1, acc, sem0, sem1):
    core = lax.axis_index("c")
    tile = lax.axis_index("s")
    row0 = tile * 640
    msg = msg0

    def one_pass(pi):
        chunk = core * 8 + pi
        base = chunk * _N

        def zfill(i):
            s = i // 8
            j = i % 8
            msg[s, pl.ds(j * 16, 16)] = jnp.zeros((16,), jnp.float32)
        _for(_K * 8, zfill)

        def zero(m):
            pltpu.sync_copy(msg, acc.at[pl.ds(row0 + m * _K, _K)])
        _for(8, zero)
        plsc.subcore_barrier()

        def win(w):
            pltpu.sync_copy(rows_hbm.at[tile, w], r2)
            pltpu.sync_copy(cols_hbm.at[tile, w], c2)

            def mask(i):
                s = i // 5
                j = i % 5
                r = r2[s, pl.ds(j * 16, 16)]
                c = c2[s, pl.ds(j * 16, 16)]
                c2[s, pl.ds(j * 16, 16)] = jnp.where(r == c, _DUMMY, c)
            _for(_WSTEPS * 5, mask)

            def start(s, ib, mb, sem):
                def fill(j):
                    ib[pl.ds(j * 16, 16)] = r2[s, pl.ds(j * 16, 16)] + base
                _for(5, fill)
                pltpu.async_copy(table_hbm.at[ib], mb, sem)

            def drain(s, ib, mb, sem):
                pltpu.make_async_copy(table_hbm.at[ib], mb, sem).wait()
                pltpu.sync_copy(mb, acc.at[c2.at[s]], add=True)

            start(0, idx0, msg0, sem0)

            def pair(t):
                s0 = 2 * t
                start(s0 + 1, idx1, msg1, sem1)
                drain(s0, idx0, msg0, sem0)
                start(s0 + 2, idx0, msg0, sem0)
                drain(s0 + 1, idx1, msg1, sem1)
            _for((_WSTEPS - 2) // 2, pair)
            start(_WSTEPS - 1, idx1, msg1, sem1)
            drain(_WSTEPS - 2, idx0, msg0, sem0)
            drain(_WSTEPS - 1, idx1, msg1, sem1)
        _for(_NWIN, win)
        plsc.subcore_barrier()

        def dump(m):
            pltpu.sync_copy(acc.at[pl.ds(row0 + m * _K, _K)], msg)
            pltpu.sync_copy(msg, s_out.at[chunk, pl.ds(row0 + m * _K, _K)])
        _for(8, dump)
    _for(8, one_pass)


def _gates_body(sp_r, up_r, dv_r, lzw_r, lzb_r, lhw_r, lhb_r,
                bz_r, bh_r, att_r, wl_r, bl_r, out_r):
    dv = dv_r[0, 0]
    S = dv[None, :, None] * (sp_r[0] + up_r[0])
    am = att_r[...]
    e = jnp.exp(am - jnp.max(am))
    probs = e / jnp.sum(e)
    acc = jnp.zeros((_TILEN, _H), jnp.float32)
    for p in range(_P):
        half, pi = p // 2, p % 2
        fs = S[half][:, pi * 64:(pi + 1) * 64]
        cz = fs[:, :_H] + bz_r[...]
        ch = fs[:, _H:] + bh_r[...]
        Z = jax.nn.sigmoid(
            jnp.dot(cz, lzw_r[: _H, :], preferred_element_type=jnp.float32)
            + lzb_r[...])
        Ht = jnp.tanh(
            jnp.dot(ch, lhw_r[: _H, :], preferred_element_type=jnp.float32)
            + lhb_r[...])
        acc = acc + probs[p:p + 1] * ((1.0 - Z) * Ht)
    res = jnp.dot(jnp.maximum(acc, 0.0), wl_r[...],
                  preferred_element_type=jnp.float32) + bl_r[...]
    out_r[0] = res


_gates_tc = pl.pallas_call(
    _gates_body,
    grid=(_B, _NBLK),
    in_specs=[
        pl.BlockSpec((1, 2, _TILEN, _F), lambda b, j: (b, 0, j, 0)),
        pl.BlockSpec((1, 2, _TILEN, _F), lambda b, j: (b, 0, j, 0)),
        pl.BlockSpec((1, 1, _TILEN), lambda b, j: (j, 0, 0)),
        pl.BlockSpec((2 * _H, _H), lambda b, j: (0, 0)),
        pl.BlockSpec((_H,), lambda b, j: (0,)),
        pl.BlockSpec((2 * _H, _H), lambda b, j: (0, 0)),
        pl.BlockSpec((_H,), lambda b, j: (0,)),
        pl.BlockSpec((_H,), lambda b, j: (0,)),
        pl.BlockSpec((_H,), lambda b, j: (0,)),
        pl.BlockSpec((_P,), lambda b, j: (0,)),
        pl.BlockSpec((_H, _P), lambda b, j: (0, 0)),
        pl.BlockSpec((_P,), lambda b, j: (0,)),
    ],
    out_specs=pl.BlockSpec((1, _TILEN, _P), lambda b, j: (b, j, 0)),
    out_shape=jax.ShapeDtypeStruct((_B, _N, _P), jnp.float32),
)


@functools.cache
def _build_sc_kernels():
    mesh = plsc.VectorSubcoreMesh(
        core_axis_name="c", subcore_axis_name="s",
        num_cores=_NC, num_subcores=_NS)
    deg = pl.kernel(
        _deg_sc_body,
        out_type=jax.ShapeDtypeStruct((_NC, _NPAD, 32), jnp.float32),
        mesh=mesh,
        scratch_types=[
            pltpu.VMEM((_AWSTEPS, _K), jnp.int32),
            pltpu.VMEM((_AWSTEPS, _K), jnp.int32),
            pltpu.VMEM((_K, 32), jnp.float32),
            pltpu.VMEM((_K, 32), jnp.float32),
            pltpu.VMEM_SHARED((_NPAD, 32), jnp.float32),
        ],
    )
    scatter = pl.kernel(
        _scatter_sc_body,
        out_type=jax.ShapeDtypeStruct((2 * _B, _NPAD, _F), jnp.float32),
        mesh=mesh,
        scratch_types=[
            pltpu.VMEM((_WSTEPS, _K), jnp.int32),
            pltpu.VMEM((_WSTEPS, _K), jnp.int32),
            pltpu.VMEM((_K,), jnp.int32),
            pltpu.VMEM((_K,), jnp.int32),
            pltpu.VMEM((_K, _F), jnp.float32),
            pltpu.VMEM((_K, _F), jnp.float32),
            pltpu.VMEM_SHARED((_NPAD, _F), jnp.float32),
            pltpu.SemaphoreType.DMA,
            pltpu.SemaphoreType.DMA,
        ],
    )
    return deg, scatter


def kernel(x, edge_index, Wz, bz, Wr, br, Wh, bh,
           LzW, Lzb, LrW, Lrb, LhW, Lhb, att, Wl, bl):
    _deg_sc, _scatter_sc = _build_sc_kernels()
    row = edge_index[0]
    col = edge_index[1]
    rows_a = row.reshape(_NC * _NS, _AWIN, _AWSTEPS, _K)
    cols_a = col.reshape(_NC * _NS, _AWIN, _AWSTEPS, _K)
    rows_c = row.reshape(_NS, _NWIN, _WSTEPS, _K)
    cols_c = col.reshape(_NS, _NWIN, _WSTEPS, _K)
    xt = jnp.transpose(x, (0, 1, 3, 2))
    Wcat = jnp.concatenate([Wz, Wh], axis=1)

    dp = _deg_sc(rows_a, cols_a)
    Up, dinv = _prep_tc(xt, dp, Wcat)
    table = Up.reshape(2 * _B * _N, _F)
    Sp = _scatter_sc(rows_c, cols_c, table)
    Sp4 = Sp.reshape(_B, 2, _NPAD, _F)
    out = _gates_tc(Sp4, Up, dinv, LzW, Lzb, LhW, Lhb,
                    bz, bh, att, Wl, bl)
    return out
